# Initial kernel scaffold; baseline (speedup 1.0000x reference)
import jax
import jax.numpy as jnp
from jax.experimental import pallas as pl


def kernel(x, table):
    raise NotImplementedError("write your pallas kernel here")


# speedup vs baseline: 1.4391x; 1.4391x over previous
import functools

import jax
import jax.numpy as jnp
from jax import lax
from jax.experimental import pallas as pl
from jax.experimental.pallas import tpu as pltpu
from jax.experimental.pallas import tpu_sc as plsc

_N = 4096 * 20
_D = 1000
_NC, _NS = 2, 16
_NW = _NC * _NS
_PER_W = _N // _NW
_K = 64
_CHUNKS = _PER_W // _K
_NBUF = 2


def _sc_gather(x_flat, table):
    mesh = plsc.VectorSubcoreMesh(core_axis_name="c", subcore_axis_name="s")

    @functools.partial(
        pl.kernel,
        mesh=mesh,
        out_type=jax.ShapeDtypeStruct((_N, _D), jnp.float32),
        compiler_params=pltpu.CompilerParams(use_tc_tiling_on_sc=False),
        scratch_types=[
            pltpu.VMEM((_NBUF, _K), jnp.int32),
            pltpu.VMEM((_NBUF, _K, _D), jnp.float32),
            pltpu.SemaphoreType.DMA,
            pltpu.SemaphoreType.DMA,
        ],
    )
    def k(idx_hbm, table_hbm, out_hbm, idx_v, rows_v, gsem, wsem):
        wid = lax.axis_index("s") * _NC + lax.axis_index("c")
        base = wid * _PER_W

        def fire(g, slot):
            off = base + g * _K
            pltpu.sync_copy(idx_hbm.at[pl.ds(off, _K)], idx_v.at[slot])
            pltpu.async_copy(table_hbm.at[idx_v.at[slot]], rows_v.at[slot],
                             gsem)

        def drain(g, slot):
            off = base + g * _K
            pltpu.make_async_copy(table_hbm.at[idx_v.at[slot]],
                                  rows_v.at[slot], gsem).wait()
            pltpu.async_copy(rows_v.at[slot], out_hbm.at[pl.ds(off, _K)],
                             wsem)

        fire(0, 0)

        def body(g, _):
            slot = lax.rem(g, _NBUF)
            nslot = lax.rem(g + 1, _NBUF)

            @pl.when(g >= 1)
            def _():
                pltpu.make_async_copy(
                    rows_v.at[nslot],
                    out_hbm.at[pl.ds(base + (g - 1) * _K, _K)],
                    wsem).wait()

            @pl.when(g + 1 < _CHUNKS)
            def _():
                fire(g + 1, nslot)

            drain(g, slot)
            return 0

        lax.fori_loop(0, _CHUNKS, body, 0)

        pltpu.make_async_copy(
            rows_v.at[lax.rem(_CHUNKS - 1, _NBUF)],
            out_hbm.at[pl.ds(base + (_CHUNKS - 1) * _K, _K)],
            wsem).wait()

    return k(x_flat, table)


def kernel(x, table):
    xf = x.reshape(-1).astype(jnp.int32)
    out = _sc_gather(xf, table)
    return out.reshape(x.shape + (table.shape[0],))


# speedup vs baseline: 1.6199x; 1.1256x over previous
import functools

import jax
import jax.numpy as jnp
from jax import lax
from jax.experimental import pallas as pl
from jax.experimental.pallas import tpu as pltpu
from jax.experimental.pallas import tpu_sc as plsc

_N = 4096 * 20
_D = 1000
_NC, _NS = 2, 16
_NW = _NC * _NS
_PER_W = _N // _NW
_K = 32
_CHUNKS = _PER_W // _K
_NBUF = 2


def _sc_gather(x_flat, table):
    mesh = plsc.VectorSubcoreMesh(core_axis_name="c", subcore_axis_name="s")

    @functools.partial(
        pl.kernel,
        mesh=mesh,
        out_type=jax.ShapeDtypeStruct((_N, _D), jnp.float32),
        compiler_params=pltpu.CompilerParams(use_tc_tiling_on_sc=False),
        scratch_types=[
            pltpu.VMEM((_NBUF, _K), jnp.int32),
            pltpu.VMEM((_NBUF, _K, _D), jnp.float32),
            pltpu.VMEM_SHARED((1000, _D), jnp.float32),
            pltpu.SemaphoreType.DMA,
            pltpu.SemaphoreType.DMA,
        ],
    )
    def k(idx_hbm, table_hbm, out_hbm, idx_v, rows_v, table_sp, gsem, wsem):
        wid = lax.axis_index("s") * _NC + lax.axis_index("c")
        base = wid * _PER_W

        @pl.when(lax.axis_index("s") == 0)
        def _():
            pltpu.sync_copy(table_hbm, table_sp)
        plsc.subcore_barrier()

        def fire(g, slot):
            off = base + g * _K
            pltpu.sync_copy(idx_hbm.at[pl.ds(off, _K)], idx_v.at[slot])
            pltpu.async_copy(table_sp.at[idx_v.at[slot]], rows_v.at[slot],
                             gsem)

        def drain(g, slot):
            off = base + g * _K
            pltpu.make_async_copy(table_sp.at[idx_v.at[slot]],
                                  rows_v.at[slot], gsem).wait()
            pltpu.async_copy(rows_v.at[slot], out_hbm.at[pl.ds(off, _K)],
                             wsem)

        fire(0, 0)

        def body(g, _):
            slot = lax.rem(g, _NBUF)
            nslot = lax.rem(g + 1, _NBUF)

            @pl.when(g >= 1)
            def _():
                pltpu.make_async_copy(
                    rows_v.at[nslot],
                    out_hbm.at[pl.ds(base + (g - 1) * _K, _K)],
                    wsem).wait()

            @pl.when(g + 1 < _CHUNKS)
            def _():
                fire(g + 1, nslot)

            drain(g, slot)
            return 0

        lax.fori_loop(0, _CHUNKS, body, 0)

        pltpu.make_async_copy(
            rows_v.at[lax.rem(_CHUNKS - 1, _NBUF)],
            out_hbm.at[pl.ds(base + (_CHUNKS - 1) * _K, _K)],
            wsem).wait()

    return k(x_flat, table)


def kernel(x, table):
    xf = x.reshape(-1).astype(jnp.int32)
    out = _sc_gather(xf, table)
    return out.reshape(x.shape + (table.shape[0],))
